# Initial kernel scaffold; baseline (speedup 1.0000x reference)
import jax
import jax.numpy as jnp
from jax.experimental import pallas as pl


def kernel(logits, u, top_k):
    raise NotImplementedError("write your pallas kernel here")


# speedup vs baseline: 5.4566x; 5.4566x over previous
import functools

import jax
import jax.numpy as jnp
from jax import lax
from jax.experimental import pallas as pl
from jax.experimental.pallas import tpu as pltpu
from jax.experimental.pallas import tpu_sc as plsc

TEMP = 0.8
K = 50
V = 100000
VPAD = 102400
NBLK = 400
NSUP = 25
NROWS = 128
NW = 32
RPW = NROWS // NW
CAP = 2048
NEG = float("-inf")
IMAX = 2**31 - 1


def _body(logits_hbm, u_hbm, probs_hbm, tok_hbm,
          rowbuf, l1, l2, gids, cvals, cpos, svals, spos,
          urow, prow, trow):
    c = lax.axis_index("c")
    s = lax.axis_index("s")
    wid = c * 16 + s
    row0 = wid * RPW

    negv = jnp.full((16,), NEG, jnp.float32)
    imaxv = jnp.full((16,), IMAX, jnp.int32)
    iota = lax.iota(jnp.int32, 16)

    def padtail(i, _):
        rowbuf[pl.ds(V + i * 16, 16)] = negv
        return 0
    lax.fori_loop(0, (VPAD - V) // 16, padtail, 0)
    pltpu.sync_copy(u_hbm, urow.at[pl.ds(0, NROWS)])

    def do_row(rr, _):
        row = row0 + rr
        pltpu.sync_copy(logits_hbm.at[pl.ds(row * V, V)],
                        rowbuf.at[pl.ds(0, V)])

        def blk(b, l2acc):
            base = b * 256
            acc = rowbuf[pl.ds(base, 16)]
            for r in range(1, 16):
                acc = jnp.maximum(acc, rowbuf[pl.ds(base + 16 * r, 16)])
            l1[pl.ds(b * 16, 16)] = acc
            l2acc = jnp.maximum(l2acc, acc)

            @pl.when(b % 16 == 15)
            def _():
                l2[pl.ds((b // 16) * 16, 16)] = l2acc

            return jnp.where(b % 16 == 15, negv, l2acc)

        lax.fori_loop(0, NBLK, blk, negv)

        def sup(si, smin):
            return jnp.minimum(smin, jnp.sort(l2[pl.ds(si * 16, 16)]))
        smin = lax.fori_loop(0, NSUP, sup,
                             jnp.full((16,), jnp.inf, jnp.float32))
        tau = smin[14]

        def scan_l1(b, cnt):
            v = l1[pl.ds(b * 16, 16)]
            m = v >= tau
            ids = b * 16 + iota
            plsc.store_compressed(gids.at[pl.ds(cnt, 16)], ids, mask=m)
            return cnt + jnp.sum(m.astype(jnp.int32))
        ng = lax.fori_loop(0, NBLK, scan_l1, jnp.int32(0))

        def cinit(i, _):
            cvals[pl.ds(i * 16, 16)] = negv
            cpos[pl.ds(i * 16, 16)] = imaxv
            return 0
        lax.fori_loop(0, (CAP + 16) // 16, cinit, 0)

        def gather(g, cnt):
            gid = gids[pl.ds(g, 16)][0]
            basep = (gid >> 4) * 256 + (gid & 15)
            p = basep + 16 * iota
            v = plsc.load_gather(rowbuf, [p])
            m = (v >= tau) & (cnt < CAP - 15)
            plsc.store_compressed(cvals.at[pl.ds(cnt, 16)], v, mask=m)
            plsc.store_compressed(cpos.at[pl.ds(cnt, 16)], p, mask=m)
            return cnt + jnp.sum(m.astype(jnp.int32))
        n = lax.fori_loop(0, ng, gather, jnp.int32(0))
        nv = (n + 15) >> 4

        def extract(i, _):
            def scan(j, st):
                bv, bp = st
                v = cvals[pl.ds(j * 16, 16)]
                p = cpos[pl.ds(j * 16, 16)]
                better = (v > bv) | ((v == bv) & (p < bp))
                return (jnp.where(better, v, bv), jnp.where(better, p, bp))
            bv, bp = lax.fori_loop(0, nv, scan, (negv, imaxv))
            m = jnp.max(bv)
            pm = jnp.min(jnp.where(bv == m, bp, IMAX))
            onehot0 = iota == 0
            plsc.store_compressed(svals.at[pl.ds(i, 16)],
                                  jnp.full((16,), m), mask=onehot0)
            plsc.store_compressed(spos.at[pl.ds(i, 16)],
                                  jnp.full((16,), pm), mask=onehot0)

            def knock(j, _):
                v = cvals[pl.ds(j * 16, 16)]
                p = cpos[pl.ds(j * 16, 16)]
                hit = (v == m) & (p == pm)
                cvals[pl.ds(j * 16, 16)] = jnp.where(hit, NEG, v)
                return 0
            lax.fori_loop(0, nv, knock, 0)
            return 0

        svals[pl.ds(48, 16)] = negv
        lax.fori_loop(0, K, extract, 0)

        dv = [svals[pl.ds(16 * j, 16)] / TEMP for j in range(4)]
        d0 = dv[0][0]
        e = [jnp.exp(dvj - d0) for dvj in dv]
        ssum = jnp.sum(e[0] + e[1] + e[2] + e[3])
        p4 = [ej / ssum for ej in e]
        uu = urow[pl.ds(row, 16)][0]
        carry = jnp.float32(0.0)
        cnt = jnp.int32(0)
        for j in range(4):
            cdf = plsc.cumsum(p4[j]) + carry
            cnt = cnt + jnp.sum((cdf < uu).astype(jnp.int32))
            carry = carry + jnp.sum(p4[j])
        sel = jnp.clip(cnt, 0, K - 1)
        tok = spos[pl.ds(sel, 16)][0]

        for j in range(4):
            prow[pl.ds(16 * j, 16)] = p4[j]
        plsc.store_compressed(trow.at[pl.ds(0, 16)],
                              jnp.full((16,), tok), mask=iota == 0)
        pltpu.sync_copy(prow, probs_hbm.at[pl.ds(row * 64, 64)])
        pltpu.sync_copy(trow.at[pl.ds(0, 8)],
                        tok_hbm.at[pl.ds(row * 8, 8)])
        return 0

    lax.fori_loop(0, RPW, do_row, 0)


@jax.jit
def _run(logits, u1):
    mesh = plsc.VectorSubcoreMesh(core_axis_name="c", subcore_axis_name="s")
    fn = pl.kernel(
        _body,
        out_type=(
            jax.ShapeDtypeStruct((NROWS * 64,), jnp.float32),
            jax.ShapeDtypeStruct((NROWS * 8,), jnp.int32),
        ),
        mesh=mesh,
        compiler_params=pltpu.CompilerParams(needs_layout_passes=False),
        scratch_types=[
            pltpu.VMEM((VPAD,), jnp.float32),
            pltpu.VMEM((NBLK * 16,), jnp.float32),
            pltpu.VMEM((NSUP * 16,), jnp.float32),
            pltpu.VMEM((NBLK * 16 + 16,), jnp.int32),
            pltpu.VMEM((CAP + 16,), jnp.float32),
            pltpu.VMEM((CAP + 16,), jnp.int32),
            pltpu.VMEM((80,), jnp.float32),
            pltpu.VMEM((80,), jnp.int32),
            pltpu.VMEM((NROWS + 16,), jnp.float32),
            pltpu.VMEM((64,), jnp.float32),
            pltpu.VMEM((16,), jnp.int32),
        ],
    )
    return fn(logits, u1)


def kernel(logits, u, top_k):
    del top_k
    probs, tok = _run(logits.reshape(NROWS * V), u.reshape(NROWS))
    return (tok.reshape(NROWS, 8)[:, :1],
            probs.reshape(NROWS, 64)[:, :K])


# speedup vs baseline: 6.1165x; 1.1209x over previous
import functools

import jax
import jax.numpy as jnp
from jax import lax
from jax.experimental import pallas as pl
from jax.experimental.pallas import tpu as pltpu
from jax.experimental.pallas import tpu_sc as plsc

TEMP = 0.8
K = 50
V = 100000
VPAD = 102400
NBLK = 400
NSUP = 25
NROWS = 128
NW = 32
RPW = NROWS // NW
CAP = 2048
WIN = 6400
NWIN = 16
NEG = float("-inf")
IMAX = 2**31 - 1


def _body(logits_hbm, u_hbm, probs_hbm, tok_hbm,
          rowbuf, l1, l2, gids, cvals, cpos, svals, spos,
          urow, prow, trow, sem0, sem1):
    sems = (sem0, sem1)
    c = lax.axis_index("c")
    s = lax.axis_index("s")
    wid = c * 16 + s
    row0 = wid * RPW

    negv = jnp.full((16,), NEG, jnp.float32)
    imaxv = jnp.full((16,), IMAX, jnp.int32)
    iota = lax.iota(jnp.int32, 16)

    def padtail(i, _):
        rowbuf[pl.ds(V + i * 16, 16)] = negv
        return 0
    lax.fori_loop(0, (VPAD - V) // 16, padtail, 0)
    pltpu.sync_copy(u_hbm, urow.at[pl.ds(0, NROWS)])

    def do_row(rr, _):
        row = row0 + rr
        base = row * V

        def dma(w):
            size = WIN if w < NWIN - 1 else V - (NWIN - 1) * WIN
            return pltpu.async_copy(
                logits_hbm.at[pl.ds(base + w * WIN, size)],
                rowbuf.at[pl.ds(w * WIN, size)],
                sems[w % 2])

        descs = [None] * NWIN
        descs[0] = dma(0)
        descs[1] = dma(1)

        for w in range(NWIN):
            descs[w].wait()
            if w + 2 < NWIN:
                descs[w + 2] = dma(w + 2)

            def blk(b2, l2acc, w=w):
                b = w * 25 + b2
                e = b * 256
                acc = rowbuf[pl.ds(e, 16)]
                for r in range(1, 16):
                    acc = jnp.maximum(acc, rowbuf[pl.ds(e + 16 * r, 16)])
                l1[pl.ds(b * 16, 16)] = acc
                return jnp.maximum(l2acc, acc)

            l2v = lax.fori_loop(0, 25, blk, negv)
            l2[pl.ds(w * 16, 16)] = l2v

        def sup(si, smin):
            return jnp.minimum(smin, jnp.sort(l2[pl.ds(si * 16, 16)]))
        smin = lax.fori_loop(0, NWIN, sup,
                             jnp.full((16,), jnp.inf, jnp.float32))
        tau = smin[12]

        def scan_l1(b, cnt):
            v = l1[pl.ds(b * 16, 16)]
            m = v >= tau
            ids = b * 16 + iota
            plsc.store_compressed(gids.at[pl.ds(cnt, 16)], ids, mask=m)
            return cnt + jnp.sum(m.astype(jnp.int32))
        ng = lax.fori_loop(0, NBLK, scan_l1, jnp.int32(0))

        def cinit(i, _):
            cvals[pl.ds(i * 16, 16)] = negv
            cpos[pl.ds(i * 16, 16)] = imaxv
            return 0
        lax.fori_loop(0, (CAP + 16) // 16, cinit, 0)

        def gather(g, cnt):
            gid = gids[pl.ds(g, 16)][0]
            basep = (gid >> 4) * 256 + (gid & 15)
            p = basep + 16 * iota
            v = plsc.load_gather(rowbuf, [p])
            m = (v >= tau) & (cnt < CAP - 15)
            plsc.store_compressed(cvals.at[pl.ds(cnt, 16)], v, mask=m)
            plsc.store_compressed(cpos.at[pl.ds(cnt, 16)], p, mask=m)
            return cnt + jnp.sum(m.astype(jnp.int32))
        n = lax.fori_loop(0, ng, gather, jnp.int32(0))
        nv = (n + 15) >> 4

        def extract(i, _):
            def scan(j, st):
                bv, bp = st
                v = cvals[pl.ds(j * 16, 16)]
                p = cpos[pl.ds(j * 16, 16)]
                better = (v > bv) | ((v == bv) & (p < bp))
                return (jnp.where(better, v, bv), jnp.where(better, p, bp))
            bv, bp = lax.fori_loop(0, nv, scan, (negv, imaxv))
            m = jnp.max(bv)
            pm = jnp.min(jnp.where(bv == m, bp, IMAX))
            onehot0 = iota == 0
            plsc.store_compressed(svals.at[pl.ds(i, 16)],
                                  jnp.full((16,), m), mask=onehot0)
            plsc.store_compressed(spos.at[pl.ds(i, 16)],
                                  jnp.full((16,), pm), mask=onehot0)

            def knock(j, _):
                v = cvals[pl.ds(j * 16, 16)]
                p = cpos[pl.ds(j * 16, 16)]
                hit = (v == m) & (p == pm)
                cvals[pl.ds(j * 16, 16)] = jnp.where(hit, NEG, v)
                return 0
            lax.fori_loop(0, nv, knock, 0)
            return 0

        svals[pl.ds(48, 16)] = negv
        lax.fori_loop(0, K, extract, 0)

        dv = [svals[pl.ds(16 * j, 16)] / TEMP for j in range(4)]
        d0 = dv[0][0]
        e = [jnp.exp(dvj - d0) for dvj in dv]
        ssum = jnp.sum(e[0] + e[1] + e[2] + e[3])
        p4 = [ej / ssum for ej in e]
        uu = urow[pl.ds(row, 16)][0]
        carry = jnp.float32(0.0)
        cnt = jnp.int32(0)
        for j in range(4):
            cdf = plsc.cumsum(p4[j]) + carry
            cnt = cnt + jnp.sum((cdf < uu).astype(jnp.int32))
            carry = carry + jnp.sum(p4[j])
        sel = jnp.clip(cnt, 0, K - 1)
        tok = spos[pl.ds(sel, 16)][0]

        for j in range(4):
            prow[pl.ds(16 * j, 16)] = p4[j]
        plsc.store_compressed(trow.at[pl.ds(0, 16)],
                              jnp.full((16,), tok), mask=iota == 0)
        pltpu.sync_copy(prow, probs_hbm.at[pl.ds(row * 64, 64)])
        pltpu.sync_copy(trow.at[pl.ds(0, 8)],
                        tok_hbm.at[pl.ds(row * 8, 8)])
        return 0

    lax.fori_loop(0, RPW, do_row, 0)


@jax.jit
def _run(logits, u1):
    mesh = plsc.VectorSubcoreMesh(core_axis_name="c", subcore_axis_name="s")
    fn = pl.kernel(
        _body,
        out_type=(
            jax.ShapeDtypeStruct((NROWS * 64,), jnp.float32),
            jax.ShapeDtypeStruct((NROWS * 8,), jnp.int32),
        ),
        mesh=mesh,
        compiler_params=pltpu.CompilerParams(needs_layout_passes=False),
        scratch_types=[
            pltpu.VMEM((VPAD,), jnp.float32),
            pltpu.VMEM((NBLK * 16,), jnp.float32),
            pltpu.VMEM((NWIN * 16,), jnp.float32),
            pltpu.VMEM((NBLK * 16 + 16,), jnp.int32),
            pltpu.VMEM((CAP + 16,), jnp.float32),
            pltpu.VMEM((CAP + 16,), jnp.int32),
            pltpu.VMEM((80,), jnp.float32),
            pltpu.VMEM((80,), jnp.int32),
            pltpu.VMEM((NROWS + 16,), jnp.float32),
            pltpu.VMEM((64,), jnp.float32),
            pltpu.VMEM((16,), jnp.int32),
            pltpu.SemaphoreType.DMA,
            pltpu.SemaphoreType.DMA,
        ],
    )
    return fn(logits, u1)


def kernel(logits, u, top_k):
    del top_k
    probs, tok = _run(logits.reshape(NROWS * V), u.reshape(NROWS))
    return (tok.reshape(NROWS, 8)[:, :1],
            probs.reshape(NROWS, 64)[:, :K])


# speedup vs baseline: 6.4210x; 1.0498x over previous
import functools

import jax
import jax.numpy as jnp
from jax import lax
from jax.experimental import pallas as pl
from jax.experimental.pallas import tpu as pltpu
from jax.experimental.pallas import tpu_sc as plsc

TEMP = 0.8
K = 50
V = 100000
VPAD = 102400
NBLK = 400
NSUP = 25
NROWS = 128
NW = 32
RPW = NROWS // NW
CAP = 2048
WIN = 6400
NWIN = 16
NEG = float("-inf")
IMAX = 2**31 - 1


def _body(logits_hbm, u_hbm, probs_hbm, tok_hbm,
          rowbuf, l1, l2, gids, cvals, cpos, svals, spos,
          urow, prow, trow, sem0, sem1):
    sems = (sem0, sem1)
    c = lax.axis_index("c")
    s = lax.axis_index("s")
    wid = c * 16 + s
    row0 = wid * RPW

    negv = jnp.full((16,), NEG, jnp.float32)
    imaxv = jnp.full((16,), IMAX, jnp.int32)
    iota = lax.iota(jnp.int32, 16)

    def padtail(i, _):
        rowbuf[pl.ds(V + i * 16, 16)] = negv
        return 0
    lax.fori_loop(0, (VPAD - V) // 16, padtail, 0)
    pltpu.sync_copy(u_hbm, urow.at[pl.ds(0, NROWS)])

    def do_row(rr, _):
        row = row0 + rr
        base = row * V

        def dma(w):
            size = WIN if w < NWIN - 1 else V - (NWIN - 1) * WIN
            return pltpu.async_copy(
                logits_hbm.at[pl.ds(base + w * WIN, size)],
                rowbuf.at[pl.ds(w * WIN, size)],
                sems[w % 2])

        descs = [None] * NWIN
        descs[0] = dma(0)
        descs[1] = dma(1)

        for w in range(NWIN):
            descs[w].wait()
            if w + 2 < NWIN:
                descs[w + 2] = dma(w + 2)

            def blk(b2, l2acc, w=w):
                b = w * 25 + b2
                e = b * 256
                acc = rowbuf[pl.ds(e, 16)]
                for r in range(1, 16):
                    acc = jnp.maximum(acc, rowbuf[pl.ds(e + 16 * r, 16)])
                l1[pl.ds(b * 16, 16)] = acc
                return jnp.maximum(l2acc, acc)

            l2v = lax.fori_loop(0, 25, blk, negv)
            l2[pl.ds(w * 16, 16)] = l2v

        def sup(si, smin):
            return jnp.minimum(smin, jnp.sort(l2[pl.ds(si * 16, 16)]))
        smin = lax.fori_loop(0, NWIN, sup,
                             jnp.full((16,), jnp.inf, jnp.float32))
        tau = smin[12]

        def scan_l1(b, cnt):
            v = l1[pl.ds(b * 16, 16)]
            m = v >= tau
            ids = b * 16 + iota
            plsc.store_compressed(gids.at[pl.ds(cnt, 16)], ids, mask=m)
            return cnt + plsc.all_reduce_population_count(m)[0]
        ng = lax.fori_loop(0, NBLK, scan_l1, jnp.int32(0))

        def gather(g, cnt):
            gid = gids[pl.ds(g, 16)][0]
            basep = (gid >> 4) * 256 + (gid & 15)
            p = basep + 16 * iota
            v = plsc.load_gather(rowbuf, [p])
            m = (v >= tau) & (cnt < CAP - 15)
            plsc.store_compressed(cvals.at[pl.ds(cnt, 16)], v, mask=m)
            plsc.store_compressed(cpos.at[pl.ds(cnt, 16)], p, mask=m)
            return cnt + plsc.all_reduce_population_count(m)[0]
        n = lax.fori_loop(0, ng, gather, jnp.int32(0))
        cvals[pl.ds(n, 16)] = negv
        cpos[pl.ds(n, 16)] = imaxv
        nv = (n + 15) >> 4

        zerov = jnp.zeros((16,), jnp.int32)

        def extract(i, _):
            def scan(j, st):
                bv, bp, bj = st
                v = cvals[pl.ds(j * 16, 16)]
                p = cpos[pl.ds(j * 16, 16)]
                better = (v > bv) | ((v == bv) & (p < bp))
                jv = jnp.full((16,), j, jnp.int32)
                return (jnp.where(better, v, bv), jnp.where(better, p, bp),
                        jnp.where(better, jv, bj))
            bv, bp, bj = lax.fori_loop(0, nv, scan, (negv, imaxv, zerov))
            m = jnp.max(bv)
            lm = bv == m
            pm = jnp.min(jnp.where(lm, bp, IMAX))
            jm = jnp.max(jnp.where(lm & (bp == pm), bj, 0))
            onehot0 = iota == 0
            plsc.store_compressed(svals.at[pl.ds(i, 16)],
                                  jnp.full((16,), m), mask=onehot0)
            plsc.store_compressed(spos.at[pl.ds(i, 16)],
                                  jnp.full((16,), pm), mask=onehot0)
            v = cvals[pl.ds(jm * 16, 16)]
            p = cpos[pl.ds(jm * 16, 16)]
            cvals[pl.ds(jm * 16, 16)] = jnp.where(p == pm, NEG, v)
            return 0

        svals[pl.ds(48, 16)] = negv
        lax.fori_loop(0, K, extract, 0)

        dv = [svals[pl.ds(16 * j, 16)] / TEMP for j in range(4)]
        d0 = dv[0][0]
        e = [jnp.exp(dvj - d0) for dvj in dv]
        ssum = jnp.sum(e[0] + e[1] + e[2] + e[3])
        p4 = [ej / ssum for ej in e]
        uu = urow[pl.ds(row, 16)][0]
        carry = jnp.float32(0.0)
        cnt = jnp.int32(0)
        for j in range(4):
            cdf = plsc.cumsum(p4[j]) + carry
            cnt = cnt + jnp.sum((cdf < uu).astype(jnp.int32))
            carry = carry + jnp.sum(p4[j])
        sel = jnp.clip(cnt, 0, K - 1)
        tok = spos[pl.ds(sel, 16)][0]

        for j in range(4):
            prow[pl.ds(16 * j, 16)] = p4[j]
        plsc.store_compressed(trow.at[pl.ds(0, 16)],
                              jnp.full((16,), tok), mask=iota == 0)
        pltpu.sync_copy(prow, probs_hbm.at[pl.ds(row * 64, 64)])
        pltpu.sync_copy(trow.at[pl.ds(0, 8)],
                        tok_hbm.at[pl.ds(row * 8, 8)])
        return 0

    lax.fori_loop(0, RPW, do_row, 0)


@jax.jit
def _run(logits, u1):
    mesh = plsc.VectorSubcoreMesh(core_axis_name="c", subcore_axis_name="s")
    fn = pl.kernel(
        _body,
        out_type=(
            jax.ShapeDtypeStruct((NROWS * 64,), jnp.float32),
            jax.ShapeDtypeStruct((NROWS * 8,), jnp.int32),
        ),
        mesh=mesh,
        compiler_params=pltpu.CompilerParams(needs_layout_passes=False),
        scratch_types=[
            pltpu.VMEM((VPAD,), jnp.float32),
            pltpu.VMEM((NBLK * 16,), jnp.float32),
            pltpu.VMEM((NWIN * 16,), jnp.float32),
            pltpu.VMEM((NBLK * 16 + 16,), jnp.int32),
            pltpu.VMEM((CAP + 16,), jnp.float32),
            pltpu.VMEM((CAP + 16,), jnp.int32),
            pltpu.VMEM((80,), jnp.float32),
            pltpu.VMEM((80,), jnp.int32),
            pltpu.VMEM((NROWS + 16,), jnp.float32),
            pltpu.VMEM((64,), jnp.float32),
            pltpu.VMEM((16,), jnp.int32),
            pltpu.SemaphoreType.DMA,
            pltpu.SemaphoreType.DMA,
        ],
    )
    return fn(logits, u1)


def kernel(logits, u, top_k):
    del top_k
    probs, tok = _run(logits.reshape(NROWS * V), u.reshape(NROWS))
    return (tok.reshape(NROWS, 8)[:, :1],
            probs.reshape(NROWS, 64)[:, :K])
